# baseline (speedup 1.0000x reference)
import functools

import jax
import jax.numpy as jnp
from jax import lax
from jax.experimental import pallas as pl
from jax.experimental.pallas import tpu as pltpu
from jax.experimental.pallas import tpu_sc as plsc

N = 1024
E = 65536
HID = 128
NCLS = 16
BN_EPS = 1e-5


_NSUB = 16
_ROWS = N // _NSUB
_EHALF = E // 2
_CHUNK = 16384
_NCH = _EHALF // _CHUNK

_mesh = plsc.VectorSubcoreMesh(core_axis_name="c", subcore_axis_name="s")


@functools.partial(
    pl.kernel,
    mesh=_mesh,
    compiler_params=pltpu.CompilerParams(use_tc_tiling_on_sc=False,
                                         needs_layout_passes=False),
    out_type=jax.ShapeDtypeStruct((2, N, N), jnp.float32),
    scratch_types=[
        pltpu.VMEM((_ROWS * N,), jnp.float32),
        pltpu.VMEM((_CHUNK,), jnp.int32),
        pltpu.VMEM((_CHUNK,), jnp.int32),
        pltpu.SemaphoreType.DMA,
    ],
)
def _build_adj(edge_hbm, zeros_hbm, out_hbm, slab, dstb, srcb, sem):
    c = lax.axis_index("c")
    s = lax.axis_index("s")
    row0 = s * _ROWS
    ebase = c * _EHALF
    pltpu.sync_copy(zeros_hbm, slab)
    ones = jnp.full((16,), 1.0, jnp.float32)
    for ch in range(_NCH):
        off = ebase + ch * _CHUNK
        pltpu.sync_copy(edge_hbm.at[1, pl.ds(off, _CHUNK)], dstb)
        pltpu.sync_copy(edge_hbm.at[0, pl.ds(off, _CHUNK)], srcb)

        @plsc.parallel_loop(0, _CHUNK // 16, unroll=8)
        def body(j):
            d = dstb[pl.ds(j * 16, 16)]
            sv = srcb[pl.ds(j * 16, 16)]
            rel = d - row0
            m = (rel >= 0) & (rel < _ROWS)
            relc = jnp.where(m, rel, 0)
            plsc.addupdate_scatter(slab, [relc * N + sv], ones, mask=m)
    handles = [
        pltpu.async_copy(slab.at[pl.ds(rr * N, N)],
                         out_hbm.at[c, row0 + rr, :], sem)
        for rr in range(_ROWS)
    ]
    for h in handles:
        h.wait()


def _dot_exact(a, b):
    return jnp.dot(a, b, preferred_element_type=jnp.float32,
                   precision=lax.Precision.HIGHEST)


def _dot_bf16(a, b):
    return jnp.dot(a.astype(jnp.bfloat16), b.astype(jnp.bfloat16),
                   preferred_element_type=jnp.float32)


def _bn_relu(z, gamma, beta):
    mu = jnp.mean(z, axis=0, keepdims=True)
    var = jnp.mean(jnp.abs(z - mu) ** 2, axis=0, keepdims=True)
    return jnp.maximum(
        gamma * (z - mu) / jnp.sqrt(var + BN_EPS) + beta, 0.0)


def _gnn_body(x_ref, a2_ref,
              w10, b10, g10, be10, w20, b20, ag0, ab0, og0, ob0,
              w11, b11, g11, be11, w21, b21, ag1, ab1, og1, ob1,
              h1_ref, h2_ref):
    A = a2_ref[0] + a2_ref[1]
    x = x_ref[...]
    z = x + _dot_exact(A, x)
    z = _dot_bf16(z, w10[...]) + b10[...]
    z = _bn_relu(z, g10[...], be10[...])
    z = _dot_bf16(z, w20[...]) + b20[...]
    z = _bn_relu(z, ag0[...], ab0[...])
    h1 = _bn_relu(z, og0[...], ob0[...])
    h1_ref[...] = h1
    z = h1 + _dot_exact(A, h1)
    z = _dot_bf16(z, w11[...]) + b11[...]
    z = _bn_relu(z, g11[...], be11[...])
    z = _dot_bf16(z, w21[...]) + b21[...]
    z = _bn_relu(z, ag1[...], ab1[...])
    h2_ref[...] = _bn_relu(z, og1[...], ob1[...])


def _gnn(x, a2, params):
    return pl.pallas_call(
        _gnn_body,
        out_shape=[jax.ShapeDtypeStruct((N, HID), jnp.float32),
                   jax.ShapeDtypeStruct((N, HID), jnp.float32)],
    )(x, a2, *params)


def _pred_body(pooled_ref, pt_ref, pb_ref, out_ref, acc_ref):
    i = pl.program_id(0)

    @pl.when(i == 0)
    def _init():
        acc_ref[...] = jnp.zeros_like(acc_ref)

    blk = pt_ref.shape[1]
    pe = pooled_ref[...].astype(jnp.bfloat16).astype(jnp.float32)
    te = pt_ref[...].astype(jnp.bfloat16).astype(jnp.float32)
    prod = pe * te
    acc_ref[...] += jnp.sum(prod.reshape(NCLS, blk // 128, 128), axis=1)

    @pl.when(i == pl.num_programs(0) - 1)
    def _fin():
        ones = jnp.ones((1, 128), jnp.float32)
        out_ref[...] = lax.dot_general(
            ones, acc_ref[...], (((1,), (1,)), ((), ())),
            preferred_element_type=jnp.float32,
            precision=lax.Precision.HIGHEST) + pb_ref[...]


def _pred(pooled, pt, pb, blk):
    K = pt.shape[1]
    return pl.pallas_call(
        _pred_body,
        grid=(K // blk,),
        in_specs=[pl.BlockSpec((1, blk), lambda i: (0, i)),
                  pl.BlockSpec((NCLS, blk), lambda i: (0, i)),
                  pl.BlockSpec((1, NCLS), lambda i: (0, 0))],
        out_specs=pl.BlockSpec((1, NCLS), lambda i: (0, 0)),
        out_shape=jax.ShapeDtypeStruct((1, NCLS), jnp.float32),
        scratch_shapes=[pltpu.VMEM((NCLS, 128), jnp.float32)],
    )(pooled, pt, pb)


def kernel(x, edge_index, layers, preds):
    (p0, pb0), (p1, pb1), (p2, pb2) = preds
    zeros = jnp.zeros((_ROWS * N,), jnp.float32)
    a2 = _build_adj(edge_index, zeros)

    r = lambda v: v.reshape(1, -1)
    s0 = _pred(x.reshape(1, -1), p0.T, r(pb0), blk=65536)

    params = []
    for lyr in layers:
        w1, b1, g1, be1, w2, b2, ag, ab, og, ob = lyr
        params += [w1, r(b1), r(g1), r(be1), w2, r(b2),
                   r(ag), r(ab), r(og), r(ob)]
    h1, h2 = _gnn(x, a2, params)

    s1 = _pred(h1.reshape(1, -1), p1.T, r(pb1), blk=32768)
    s2 = _pred(h2.reshape(1, -1), p2.T, r(pb2), blk=32768)
    return s0 + s1 + s2


# speedup vs baseline: 1.0788x; 1.0788x over previous
import functools

import jax
import jax.numpy as jnp
from jax import lax
from jax.experimental import pallas as pl
from jax.experimental.pallas import tpu as pltpu
from jax.experimental.pallas import tpu_sc as plsc

N = 1024
E = 65536
HID = 128
NCLS = 16
BN_EPS = 1e-5


_NSUB = 16
_ROWS = N // _NSUB
_EHALF = E // 2
_CHUNK = 16384
_NCH = _EHALF // _CHUNK

_mesh = plsc.VectorSubcoreMesh(core_axis_name="c", subcore_axis_name="s")


@functools.partial(
    pl.kernel,
    mesh=_mesh,
    compiler_params=pltpu.CompilerParams(use_tc_tiling_on_sc=False,
                                         needs_layout_passes=False),
    out_type=jax.ShapeDtypeStruct((2, N, N), jnp.float32),
    scratch_types=[
        pltpu.VMEM((_ROWS * N,), jnp.float32),
        pltpu.VMEM((_CHUNK,), jnp.int32),
        pltpu.VMEM((_CHUNK,), jnp.int32),
        pltpu.SemaphoreType.DMA,
    ],
)
def _build_adj(edge_hbm, zeros_hbm, out_hbm, slab, dstb, srcb, sem):
    c = lax.axis_index("c")
    s = lax.axis_index("s")
    row0 = s * _ROWS
    ebase = c * _EHALF
    pltpu.sync_copy(zeros_hbm, slab)
    ones = jnp.full((16,), 1.0, jnp.float32)
    for ch in range(_NCH):
        off = ebase + ch * _CHUNK
        pltpu.sync_copy(edge_hbm.at[1, pl.ds(off, _CHUNK)], dstb)
        pltpu.sync_copy(edge_hbm.at[0, pl.ds(off, _CHUNK)], srcb)

        @plsc.parallel_loop(0, _CHUNK // 16, unroll=8)
        def body(j):
            d = dstb[pl.ds(j * 16, 16)]
            sv = srcb[pl.ds(j * 16, 16)]
            rel = d - row0
            m = (rel >= 0) & (rel < _ROWS)
            relc = jnp.where(m, rel, 0)
            plsc.addupdate_scatter(slab, [relc * N + sv], ones, mask=m)
    handles = [
        pltpu.async_copy(slab.at[pl.ds(rr * N, N)],
                         out_hbm.at[c, row0 + rr, :], sem)
        for rr in range(_ROWS)
    ]
    for h in handles:
        h.wait()


def _dot_exact(a, b):
    ab = a.astype(jnp.bfloat16)
    hi = b.astype(jnp.bfloat16)
    r1 = b - hi.astype(jnp.float32)
    mid = r1.astype(jnp.bfloat16)
    lo = (r1 - mid.astype(jnp.float32)).astype(jnp.bfloat16)
    f32 = jnp.float32
    return (jnp.dot(ab, hi, preferred_element_type=f32)
            + jnp.dot(ab, mid, preferred_element_type=f32)
            + jnp.dot(ab, lo, preferred_element_type=f32))


def _dot_bf16(a, b):
    return jnp.dot(a.astype(jnp.bfloat16), b.astype(jnp.bfloat16),
                   preferred_element_type=jnp.float32)


def _bn_relu(z, gamma, beta):
    mu = jnp.mean(z, axis=0, keepdims=True)
    var = jnp.mean(jnp.abs(z - mu) ** 2, axis=0, keepdims=True)
    return jnp.maximum(
        gamma * (z - mu) / jnp.sqrt(var + BN_EPS) + beta, 0.0)


def _gnn_body(x_ref, a2_ref,
              w10, b10, g10, be10, w20, b20, ag0, ab0, og0, ob0,
              w11, b11, g11, be11, w21, b21, ag1, ab1, og1, ob1,
              h1_ref, h2_ref):
    A = a2_ref[0] + a2_ref[1]
    x = x_ref[...]
    z = x + _dot_exact(A, x)
    z = _dot_bf16(z, w10[...]) + b10[...]
    z = _bn_relu(z, g10[...], be10[...])
    z = _dot_bf16(z, w20[...]) + b20[...]
    z = _bn_relu(z, ag0[...], ab0[...])
    h1 = _bn_relu(z, og0[...], ob0[...])
    h1_ref[...] = h1
    z = h1 + _dot_exact(A, h1)
    z = _dot_bf16(z, w11[...]) + b11[...]
    z = _bn_relu(z, g11[...], be11[...])
    z = _dot_bf16(z, w21[...]) + b21[...]
    z = _bn_relu(z, ag1[...], ab1[...])
    h2_ref[...] = _bn_relu(z, og1[...], ob1[...])


def _gnn(x, a2, params):
    return pl.pallas_call(
        _gnn_body,
        out_shape=[jax.ShapeDtypeStruct((N, HID), jnp.float32),
                   jax.ShapeDtypeStruct((N, HID), jnp.float32)],
    )(x, a2, *params)


def _pred_body(pooled_ref, pt_ref, pb_ref, out_ref, acc_ref):
    i = pl.program_id(0)

    @pl.when(i == 0)
    def _init():
        acc_ref[...] = jnp.zeros_like(acc_ref)

    blk = pt_ref.shape[1]
    pe = pooled_ref[...].astype(jnp.bfloat16).astype(jnp.float32)
    te = pt_ref[...].astype(jnp.bfloat16).astype(jnp.float32)
    prod = pe * te
    acc_ref[...] += jnp.sum(prod.reshape(NCLS, blk // 128, 128), axis=1)

    @pl.when(i == pl.num_programs(0) - 1)
    def _fin():
        ones = jnp.ones((1, 128), jnp.float32)
        out_ref[...] = lax.dot_general(
            ones, acc_ref[...], (((1,), (1,)), ((), ())),
            preferred_element_type=jnp.float32,
            precision=lax.Precision.HIGHEST) + pb_ref[...]


def _pred(pooled, pt, pb, blk):
    K = pt.shape[1]
    return pl.pallas_call(
        _pred_body,
        grid=(K // blk,),
        in_specs=[pl.BlockSpec((1, blk), lambda i: (0, i)),
                  pl.BlockSpec((NCLS, blk), lambda i: (0, i)),
                  pl.BlockSpec((1, NCLS), lambda i: (0, 0))],
        out_specs=pl.BlockSpec((1, NCLS), lambda i: (0, 0)),
        out_shape=jax.ShapeDtypeStruct((1, NCLS), jnp.float32),
        scratch_shapes=[pltpu.VMEM((NCLS, 128), jnp.float32)],
    )(pooled, pt, pb)


def kernel(x, edge_index, layers, preds):
    (p0, pb0), (p1, pb1), (p2, pb2) = preds
    zeros = jnp.zeros((_ROWS * N,), jnp.float32)
    a2 = _build_adj(edge_index, zeros)

    r = lambda v: v.reshape(1, -1)
    s0 = _pred(x.reshape(1, -1), p0.T, r(pb0), blk=65536)

    params = []
    for lyr in layers:
        w1, b1, g1, be1, w2, b2, ag, ab, og, ob = lyr
        params += [w1, r(b1), r(g1), r(be1), w2, r(b2),
                   r(ag), r(ab), r(og), r(ob)]
    h1, h2 = _gnn(x, a2, params)

    s1 = _pred(h1.reshape(1, -1), p1.T, r(pb1), blk=32768)
    s2 = _pred(h2.reshape(1, -1), p2.T, r(pb2), blk=32768)
    return s0 + s1 + s2
